# baseline (speedup 1.0000x reference)
import functools

import jax
import jax.numpy as jnp
from jax import lax
from jax.experimental import pallas as pl
from jax.experimental.pallas import tpu as pltpu
from jax.experimental.pallas import tpu_sc as plsc

N = 4096
TL = 50
BL = 200
D = 64
V = 100000
C = 1000
NW = 32
CH = N // NW


def _zero_acc(acc):
    zero = jnp.zeros((16,), jnp.float32)

    def body(e, _):
        acc[e, pl.ds(0, 16)] = zero
        acc[e, pl.ds(16, 16)] = zero
        acc[e, pl.ds(32, 16)] = zero
        acc[e, pl.ds(48, 16)] = zero
        return 0

    lax.fori_loop(0, CH, body, 0)


def _sc_pool_body(w_hbm, title_hbm, body_hbm, que_hbm,
                  tidx_u, bidx_u, tidx_v, bidx_v, acc_t, acc_b,
                  cnt_t, cnt_b, queT, sem_t, sem_b):
    wid = lax.axis_index("s") * 2 + lax.axis_index("c")
    base = wid * CH

    pltpu.sync_copy(title_hbm.at[pl.ds(base * TL, CH * TL)], tidx_u)
    pltpu.sync_copy(body_hbm.at[pl.ds(base * BL, CH * BL)], bidx_u)
    _zero_acc(acc_t)
    _zero_acc(acc_b)

    zf = jnp.zeros((16,), jnp.float32)
    onef = jnp.ones((16,), jnp.float32)
    for e0 in range(CH // 16):
        cnt_t[pl.ds(e0 * 16, 16)] = zf
        cnt_b[pl.ds(e0 * 16, 16)] = zf

    lane = lax.iota(jnp.int32, 16)

    def tpass(k, _):
        for e0 in range(CH // 16):
            flat = (e0 * 16 + lane) * TL + k
            v = plsc.load_gather(tidx_u, [flat])
            tidx_v[k, pl.ds(e0 * 16, 16)] = v
            cnt_t[pl.ds(e0 * 16, 16)] = (cnt_t[pl.ds(e0 * 16, 16)]
                                         + jnp.where(v > 0, onef, zf))
        pltpu.async_copy(w_hbm.at[tidx_v.at[k]], acc_t, sem_t, add=True)
        return 0

    def bpass(k, _):
        for e0 in range(CH // 16):
            flat = (e0 * 16 + lane) * BL + k
            v = plsc.load_gather(bidx_u, [flat])
            bidx_v[k, pl.ds(e0 * 16, 16)] = v
            cnt_b[pl.ds(e0 * 16, 16)] = (cnt_b[pl.ds(e0 * 16, 16)]
                                         + jnp.where(v > 0, onef, zf))
        pltpu.async_copy(w_hbm.at[bidx_v.at[k]], acc_b, sem_b, add=True)
        return 0

    lax.fori_loop(0, TL, tpass, 0)
    lax.fori_loop(0, BL, bpass, 0)

    for e0 in range(CH // 16):
        cnt_t[pl.ds(e0 * 16, 16)] = 0.3 / cnt_t[pl.ds(e0 * 16, 16)]
        cnt_b[pl.ds(e0 * 16, 16)] = 0.7 / cnt_b[pl.ds(e0 * 16, 16)]

    def tdrain(k, _):
        pltpu.make_async_copy(w_hbm.at[tidx_v.at[0]], acc_t, sem_t).wait()
        return 0

    def bdrain(k, _):
        pltpu.make_async_copy(w_hbm.at[bidx_v.at[0]], acc_b, sem_b).wait()
        return 0

    lax.fori_loop(0, TL, tdrain, 0)
    lax.fori_loop(0, BL, bdrain, 0)

    def nform(d, _):
        cold = jnp.full((16,), 0, jnp.int32) + d
        for e0 in range(CH // 16):
            rows = e0 * 16 + lane
            tg = plsc.load_gather(acc_t, [rows, cold])
            bg = plsc.load_gather(acc_b, [rows, cold])
            st = cnt_t[pl.ds(e0 * 16, 16)]
            sb = cnt_b[pl.ds(e0 * 16, 16)]
            queT[d, pl.ds(e0 * 16, 16)] = tg * st + bg * sb
        return 0

    lax.fori_loop(0, D, nform, 0)
    pltpu.sync_copy(queT, que_hbm.at[wid])


_sc_pool = functools.partial(
    pl.kernel,
    out_type=jax.ShapeDtypeStruct((NW, D, CH), jnp.float32),
    mesh=plsc.VectorSubcoreMesh(core_axis_name="c", subcore_axis_name="s"),
    scratch_types=[
        pltpu.VMEM((CH * TL,), jnp.int32),
        pltpu.VMEM((CH * BL,), jnp.int32),
        pltpu.VMEM((TL, CH), jnp.int32),
        pltpu.VMEM((BL, CH), jnp.int32),
        pltpu.VMEM((CH, D), jnp.float32),
        pltpu.VMEM((CH, D), jnp.float32),
        pltpu.VMEM((CH,), jnp.float32),
        pltpu.VMEM((CH,), jnp.float32),
        pltpu.VMEM((D, CH), jnp.float32),
        pltpu.SemaphoreType.DMA,
        pltpu.SemaphoreType.DMA,
    ],
    compiler_params=pltpu.CompilerParams(use_tc_tiling_on_sc=False,
                                         needs_layout_passes=False),
)(_sc_pool_body)


def _head_body(q_ref, c_ref, o_ref):
    q = jnp.transpose(q_ref[...], (1, 0, 2)).reshape(D, _R)
    sc = lax.dot_general(c_ref[...], q, (((1,), (0,)), ((), ())),
                         preferred_element_type=jnp.float32)
    m = jnp.max(sc, axis=0, keepdims=True)
    e = jnp.exp(sc - m)
    o_ref[...] = e / jnp.sum(e, axis=0, keepdims=True)


_R = 512


def _head(que3, c_table):
    return pl.pallas_call(
        _head_body,
        out_shape=jax.ShapeDtypeStruct((C, N), jnp.float32),
        grid=(N // _R,),
        in_specs=[
            pl.BlockSpec((_R // CH, D, CH), lambda i: (i, 0, 0)),
            pl.BlockSpec((C, D), lambda i: (0, 0)),
        ],
        out_specs=pl.BlockSpec((C, _R), lambda i: (0, i)),
    )(que3, c_table)


def kernel(title_int, body_int, user_int, w_table, c_table):
    t = title_int.astype(jnp.int32)
    b = body_int.astype(jnp.int32)
    que3 = _sc_pool(w_table, t.reshape(-1), b.reshape(-1))
    return _head(que3, c_table).T


# speedup vs baseline: 1.0021x; 1.0021x over previous
import functools

import jax
import jax.numpy as jnp
from jax import lax
from jax.experimental import pallas as pl
from jax.experimental.pallas import tpu as pltpu
from jax.experimental.pallas import tpu_sc as plsc

N = 4096
TL = 50
BL = 200
D = 64
V = 100000
C = 1000
NW = 32
CH = N // NW


def _zero_acc(acc):
    zero = jnp.zeros((16,), jnp.float32)

    def body(e, _):
        acc[e, pl.ds(0, 16)] = zero
        acc[e, pl.ds(16, 16)] = zero
        acc[e, pl.ds(32, 16)] = zero
        acc[e, pl.ds(48, 16)] = zero
        return 0

    lax.fori_loop(0, CH, body, 0)


def _sc_pool_body(w_hbm, title_hbm, body_hbm, que_hbm,
                  tidx_u, bidx_u, tidx_v, bidx_v, acc_t, acc_b,
                  cnt_t, cnt_b, queT, sem_t, sem_b):
    wid = lax.axis_index("s") * 2 + lax.axis_index("c")
    base = wid * CH

    pltpu.sync_copy(title_hbm.at[pl.ds(base * TL, CH * TL)], tidx_u)
    pltpu.sync_copy(body_hbm.at[pl.ds(base * BL, CH * BL)], bidx_u)
    _zero_acc(acc_t)
    _zero_acc(acc_b)

    zf = jnp.zeros((16,), jnp.float32)
    onef = jnp.ones((16,), jnp.float32)
    for e0 in range(CH // 16):
        cnt_t[pl.ds(e0 * 16, 16)] = zf
        cnt_b[pl.ds(e0 * 16, 16)] = zf

    lane = lax.iota(jnp.int32, 16)

    def tpass(k, _):
        for e0 in range(CH // 16):
            flat = (e0 * 16 + lane) * TL + k
            tidx_v[k, pl.ds(e0 * 16, 16)] = plsc.load_gather(tidx_u, [flat])
        pltpu.async_copy(w_hbm.at[tidx_v.at[k]], acc_t, sem_t, add=True)
        return 0

    def bpass(k, _):
        for e0 in range(CH // 16):
            flat = (e0 * 16 + lane) * BL + k
            bidx_v[k, pl.ds(e0 * 16, 16)] = plsc.load_gather(bidx_u, [flat])
        pltpu.async_copy(w_hbm.at[bidx_v.at[k]], acc_b, sem_b, add=True)
        return 0

    lax.fori_loop(0, TL, tpass, 0)
    lax.fori_loop(0, BL, bpass, 0)

    def tcnt(k, _):
        for e0 in range(CH // 16):
            v = tidx_v[k, pl.ds(e0 * 16, 16)]
            cnt_t[pl.ds(e0 * 16, 16)] = (cnt_t[pl.ds(e0 * 16, 16)]
                                         + jnp.where(v > 0, onef, zf))
        return 0

    def bcnt(k, _):
        for e0 in range(CH // 16):
            v = bidx_v[k, pl.ds(e0 * 16, 16)]
            cnt_b[pl.ds(e0 * 16, 16)] = (cnt_b[pl.ds(e0 * 16, 16)]
                                         + jnp.where(v > 0, onef, zf))
        return 0

    lax.fori_loop(0, TL, tcnt, 0)
    lax.fori_loop(0, BL, bcnt, 0)
    for e0 in range(CH // 16):
        cnt_t[pl.ds(e0 * 16, 16)] = 0.3 / cnt_t[pl.ds(e0 * 16, 16)]
        cnt_b[pl.ds(e0 * 16, 16)] = 0.7 / cnt_b[pl.ds(e0 * 16, 16)]

    def tdrain(k, _):
        pltpu.make_async_copy(w_hbm.at[tidx_v.at[0]], acc_t, sem_t).wait()
        return 0

    def bdrain(k, _):
        pltpu.make_async_copy(w_hbm.at[bidx_v.at[0]], acc_b, sem_b).wait()
        return 0

    lax.fori_loop(0, TL, tdrain, 0)
    lax.fori_loop(0, BL, bdrain, 0)

    def nform(d, _):
        cold = jnp.full((16,), 0, jnp.int32) + d
        for e0 in range(CH // 16):
            rows = e0 * 16 + lane
            tg = plsc.load_gather(acc_t, [rows, cold])
            bg = plsc.load_gather(acc_b, [rows, cold])
            st = cnt_t[pl.ds(e0 * 16, 16)]
            sb = cnt_b[pl.ds(e0 * 16, 16)]
            queT[d, pl.ds(e0 * 16, 16)] = tg * st + bg * sb
        return 0

    lax.fori_loop(0, D, nform, 0)
    pltpu.sync_copy(queT, que_hbm.at[wid])


_sc_pool = functools.partial(
    pl.kernel,
    out_type=jax.ShapeDtypeStruct((NW, D, CH), jnp.float32),
    mesh=plsc.VectorSubcoreMesh(core_axis_name="c", subcore_axis_name="s"),
    scratch_types=[
        pltpu.VMEM((CH * TL,), jnp.int32),
        pltpu.VMEM((CH * BL,), jnp.int32),
        pltpu.VMEM((TL, CH), jnp.int32),
        pltpu.VMEM((BL, CH), jnp.int32),
        pltpu.VMEM((CH, D), jnp.float32),
        pltpu.VMEM((CH, D), jnp.float32),
        pltpu.VMEM((CH,), jnp.float32),
        pltpu.VMEM((CH,), jnp.float32),
        pltpu.VMEM((D, CH), jnp.float32),
        pltpu.SemaphoreType.DMA,
        pltpu.SemaphoreType.DMA,
    ],
    compiler_params=pltpu.CompilerParams(use_tc_tiling_on_sc=False,
                                         needs_layout_passes=False),
)(_sc_pool_body)


def _head_body(q_ref, c_ref, o_ref):
    q = jnp.transpose(q_ref[...], (1, 0, 2)).reshape(D, _R)
    sc = lax.dot_general(c_ref[...], q, (((1,), (0,)), ((), ())),
                         preferred_element_type=jnp.float32)
    m = jnp.max(sc, axis=0, keepdims=True)
    e = jnp.exp(sc - m)
    o_ref[...] = e / jnp.sum(e, axis=0, keepdims=True)


_R = 512


def _head(que3, c_table):
    return pl.pallas_call(
        _head_body,
        out_shape=jax.ShapeDtypeStruct((C, N), jnp.float32),
        grid=(N // _R,),
        in_specs=[
            pl.BlockSpec((_R // CH, D, CH), lambda i: (i, 0, 0)),
            pl.BlockSpec((C, D), lambda i: (0, 0)),
        ],
        out_specs=pl.BlockSpec((C, _R), lambda i: (0, i)),
    )(que3, c_table)


def kernel(title_int, body_int, user_int, w_table, c_table):
    t = title_int.astype(jnp.int32)
    b = body_int.astype(jnp.int32)
    que3 = _sc_pool(w_table, t.reshape(-1), b.reshape(-1))
    return _head(que3, c_table).T


# speedup vs baseline: 1.0077x; 1.0056x over previous
import functools

import jax
import jax.numpy as jnp
from jax import lax
from jax.experimental import pallas as pl
from jax.experimental.pallas import tpu as pltpu
from jax.experimental.pallas import tpu_sc as plsc

N = 4096
TL = 50
BL = 200
D = 64
V = 100000
C = 1000
NW = 32
CH = N // NW


def _zero_acc(acc):
    zero = jnp.zeros((16,), jnp.float32)

    def body(e, _):
        acc[e, pl.ds(0, 16)] = zero
        acc[e, pl.ds(16, 16)] = zero
        acc[e, pl.ds(32, 16)] = zero
        acc[e, pl.ds(48, 16)] = zero
        return 0

    lax.fori_loop(0, CH, body, 0)


def _sc_pool_body(w_hbm, title_hbm, body_hbm, tsum_hbm, bsum_hbm,
                  tidx_u, bidx_u, tidx_v, bidx_v,
                  acc_t, acc_t1, acc_b, acc_b1, sem_t, sem_b):
    wid = lax.axis_index("s") * 2 + lax.axis_index("c")
    base = wid * CH

    pltpu.sync_copy(title_hbm.at[pl.ds(base * TL, CH * TL)], tidx_u)
    pltpu.sync_copy(body_hbm.at[pl.ds(base * BL, CH * BL)], bidx_u)
    _zero_acc(acc_t)
    _zero_acc(acc_t1)
    _zero_acc(acc_b)
    _zero_acc(acc_b1)

    lane = lax.iota(jnp.int32, 16)

    def tpass(k, acc):
        for e0 in range(CH // 16):
            flat = (e0 * 16 + lane) * TL + k
            tidx_v[k, pl.ds(e0 * 16, 16)] = plsc.load_gather(tidx_u, [flat])
        pltpu.async_copy(w_hbm.at[tidx_v.at[k]], acc, sem_t, add=True)

    def bpass(k, acc):
        for e0 in range(CH // 16):
            flat = (e0 * 16 + lane) * BL + k
            bidx_v[k, pl.ds(e0 * 16, 16)] = plsc.load_gather(bidx_u, [flat])
        pltpu.async_copy(w_hbm.at[bidx_v.at[k]], acc, sem_b, add=True)

    def fire(i, _):
        bpass(2 * i, acc_b)
        bpass(2 * i + 1, acc_b1)

        @pl.when(i < TL // 2)
        def _():
            tpass(2 * i, acc_t)
            tpass(2 * i + 1, acc_t1)

        return 0

    lax.fori_loop(0, BL // 2, fire, 0)

    def tdrain(k, _):
        pltpu.make_async_copy(w_hbm.at[tidx_v.at[0]], acc_t, sem_t).wait()
        return 0

    def bdrain(k, _):
        pltpu.make_async_copy(w_hbm.at[bidx_v.at[0]], acc_b, sem_b).wait()
        return 0

    lax.fori_loop(0, TL, tdrain, 0)
    lax.fori_loop(0, BL, bdrain, 0)

    def combine(e, _):
        for d in range(D // 16):
            s = pl.ds(d * 16, 16)
            acc_t[e, s] = acc_t[e, s] + acc_t1[e, s]
            acc_b[e, s] = acc_b[e, s] + acc_b1[e, s]
        return 0

    lax.fori_loop(0, CH, combine, 0)

    pltpu.sync_copy(acc_t, tsum_hbm.at[pl.ds(base, CH)])
    pltpu.sync_copy(acc_b, bsum_hbm.at[pl.ds(base, CH)])


_sc_pool = functools.partial(
    pl.kernel,
    out_type=(
        jax.ShapeDtypeStruct((N, D), jnp.float32),
        jax.ShapeDtypeStruct((N, D), jnp.float32),
    ),
    mesh=plsc.VectorSubcoreMesh(core_axis_name="c", subcore_axis_name="s"),
    scratch_types=[
        pltpu.VMEM((CH * TL,), jnp.int32),
        pltpu.VMEM((CH * BL,), jnp.int32),
        pltpu.VMEM((TL, CH), jnp.int32),
        pltpu.VMEM((BL, CH), jnp.int32),
        pltpu.VMEM((CH, D), jnp.float32),
        pltpu.VMEM((CH, D), jnp.float32),
        pltpu.VMEM((CH, D), jnp.float32),
        pltpu.VMEM((CH, D), jnp.float32),
        pltpu.SemaphoreType.DMA,
        pltpu.SemaphoreType.DMA,
    ],
    compiler_params=pltpu.CompilerParams(use_tc_tiling_on_sc=False,
                                         needs_layout_passes=False),
)(_sc_pool_body)


def _head_body(tidx_ref, bidx_ref, ts_ref, bs_ref, c_ref, o_ref):
    tcnt = jnp.sum((tidx_ref[...] > 0).astype(jnp.float32), axis=1, keepdims=True)
    bcnt = jnp.sum((bidx_ref[...] > 0).astype(jnp.float32), axis=1, keepdims=True)
    que = 0.3 * ts_ref[...] / tcnt + 0.7 * bs_ref[...] / bcnt
    sc = lax.dot_general(c_ref[...], que, (((1,), (1,)), ((), ())),
                         preferred_element_type=jnp.float32)
    m = jnp.max(sc, axis=0, keepdims=True)
    e = jnp.exp(sc - m)
    o_ref[...] = e / jnp.sum(e, axis=0, keepdims=True)


_R = 512


def _head(tidx, bidx, tsum, bsum, c_table):
    return pl.pallas_call(
        _head_body,
        out_shape=jax.ShapeDtypeStruct((C, N), jnp.float32),
        grid=(N // _R,),
        in_specs=[
            pl.BlockSpec((_R, TL), lambda i: (i, 0)),
            pl.BlockSpec((_R, BL), lambda i: (i, 0)),
            pl.BlockSpec((_R, D), lambda i: (i, 0)),
            pl.BlockSpec((_R, D), lambda i: (i, 0)),
            pl.BlockSpec((C, D), lambda i: (0, 0)),
        ],
        out_specs=pl.BlockSpec((C, _R), lambda i: (0, i)),
    )(tidx, bidx, tsum, bsum, c_table)


def kernel(title_int, body_int, user_int, w_table, c_table):
    t = title_int.astype(jnp.int32)
    b = body_int.astype(jnp.int32)
    tsum, bsum = _sc_pool(w_table, t.reshape(-1), b.reshape(-1))
    return _head(t, b, tsum, bsum, c_table).T


# speedup vs baseline: 1.0241x; 1.0163x over previous
import functools

import jax
import jax.numpy as jnp
from jax import lax
from jax.experimental import pallas as pl
from jax.experimental.pallas import tpu as pltpu
from jax.experimental.pallas import tpu_sc as plsc

N = 4096
TL = 50
BL = 200
D = 64
V = 100000
C = 1000
NW = 32
CH = N // NW


def _zero_acc(acc):
    zero = jnp.zeros((16,), jnp.float32)

    def body(e, _):
        acc[e, pl.ds(0, 16)] = zero
        acc[e, pl.ds(16, 16)] = zero
        acc[e, pl.ds(32, 16)] = zero
        acc[e, pl.ds(48, 16)] = zero
        return 0

    lax.fori_loop(0, CH, body, 0)


def _sc_pool_body(w_hbm, title_hbm, body_hbm, tsum_hbm, bsum_hbm,
                  tidx_u, bidx_u, tidx_v, bidx_v, acc_t, acc_b, sem_t, sem_b):
    wid = lax.axis_index("s") * 2 + lax.axis_index("c")
    base = wid * CH

    pltpu.sync_copy(title_hbm.at[pl.ds(base * TL, CH * TL)], tidx_u)
    pltpu.sync_copy(body_hbm.at[pl.ds(base * BL, CH * BL)], bidx_u)
    _zero_acc(acc_t)
    _zero_acc(acc_b)

    lane = lax.iota(jnp.int32, 16)

    def tpass(k, _):
        for e0 in range(CH // 16):
            flat = (e0 * 16 + lane) * TL + k
            tidx_v[k, pl.ds(e0 * 16, 16)] = plsc.load_gather(tidx_u, [flat])
        pltpu.async_copy(w_hbm.at[tidx_v.at[k]], acc_t, sem_t, add=True)
        return 0

    def bpass(k, _):
        for e0 in range(CH // 16):
            flat = (e0 * 16 + lane) * BL + k
            bidx_v[k, pl.ds(e0 * 16, 16)] = plsc.load_gather(bidx_u, [flat])
        pltpu.async_copy(w_hbm.at[bidx_v.at[k]], acc_b, sem_b, add=True)
        return 0

    lax.fori_loop(0, TL, tpass, 0)
    lax.fori_loop(0, BL, bpass, 0)

    def tdrain(k, _):
        pltpu.make_async_copy(w_hbm.at[tidx_v.at[0]], acc_t, sem_t).wait()
        return 0

    def bdrain(k, _):
        pltpu.make_async_copy(w_hbm.at[bidx_v.at[0]], acc_b, sem_b).wait()
        return 0

    lax.fori_loop(0, TL, tdrain, 0)
    lax.fori_loop(0, BL, bdrain, 0)

    pltpu.sync_copy(acc_t, tsum_hbm.at[pl.ds(base, CH)])
    pltpu.sync_copy(acc_b, bsum_hbm.at[pl.ds(base, CH)])


_sc_pool = functools.partial(
    pl.kernel,
    out_type=(
        jax.ShapeDtypeStruct((N, D), jnp.float32),
        jax.ShapeDtypeStruct((N, D), jnp.float32),
    ),
    mesh=plsc.VectorSubcoreMesh(core_axis_name="c", subcore_axis_name="s"),
    scratch_types=[
        pltpu.VMEM((CH * TL,), jnp.int32),
        pltpu.VMEM((CH * BL,), jnp.int32),
        pltpu.VMEM((TL, CH), jnp.int32),
        pltpu.VMEM((BL, CH), jnp.int32),
        pltpu.VMEM((CH, D), jnp.float32),
        pltpu.VMEM((CH, D), jnp.float32),
        pltpu.SemaphoreType.DMA,
        pltpu.SemaphoreType.DMA,
    ],
    compiler_params=pltpu.CompilerParams(use_tc_tiling_on_sc=False,
                                         needs_layout_passes=False),
)(_sc_pool_body)


def _head_body(tidx_ref, bidx_ref, ts_ref, bs_ref, c_ref, o_ref):
    tcnt = jnp.sum((tidx_ref[...] > 0).astype(jnp.float32), axis=1, keepdims=True)
    bcnt = jnp.sum((bidx_ref[...] > 0).astype(jnp.float32), axis=1, keepdims=True)
    que = 0.3 * ts_ref[...] / tcnt + 0.7 * bs_ref[...] / bcnt
    sc = lax.dot_general(c_ref[...], que, (((1,), (1,)), ((), ())),
                         preferred_element_type=jnp.float32)
    m = jnp.max(sc, axis=0, keepdims=True)
    e = jnp.exp(sc - m)
    o_ref[...] = e / jnp.sum(e, axis=0, keepdims=True)


_R = 512


def _head(tidx, bidx, tsum, bsum, c_table):
    return pl.pallas_call(
        _head_body,
        out_shape=jax.ShapeDtypeStruct((C, N), jnp.float32),
        grid=(N // _R,),
        in_specs=[
            pl.BlockSpec((_R, TL), lambda i: (i, 0)),
            pl.BlockSpec((_R, BL), lambda i: (i, 0)),
            pl.BlockSpec((_R, D), lambda i: (i, 0)),
            pl.BlockSpec((_R, D), lambda i: (i, 0)),
            pl.BlockSpec((C, D), lambda i: (0, 0)),
        ],
        out_specs=pl.BlockSpec((C, _R), lambda i: (0, i)),
    )(tidx, bidx, tsum, bsum, c_table)


def kernel(title_int, body_int, user_int, w_table, c_table):
    t = title_int.astype(jnp.int32)
    b = body_int.astype(jnp.int32)
    tsum, bsum = _sc_pool(w_table, t.reshape(-1), b.reshape(-1))
    return _head(t, b, tsum, bsum, c_table).T


# speedup vs baseline: 1.0357x; 1.0113x over previous
import functools

import jax
import jax.numpy as jnp
from jax import lax
from jax.experimental import pallas as pl
from jax.experimental.pallas import tpu as pltpu
from jax.experimental.pallas import tpu_sc as plsc

N = 4096
TL = 50
BL = 200
D = 64
V = 100000
C = 1000
NW = 32
CH = N // NW


def _zero_acc(acc):
    zero = jnp.zeros((16,), jnp.float32)

    def body(e, _):
        acc[e, pl.ds(0, 16)] = zero
        acc[e, pl.ds(16, 16)] = zero
        acc[e, pl.ds(32, 16)] = zero
        acc[e, pl.ds(48, 16)] = zero
        return 0

    lax.fori_loop(0, CH, body, 0)


def _sc_pool_body(w_hbm, title_hbm, body_hbm, tsum_hbm, bsum_hbm,
                  tidx_u, bidx_u, tidx_v, bidx_v, acc_t, acc_b, sem_t, sem_b):
    wid = lax.axis_index("s") * 2 + lax.axis_index("c")
    base = wid * CH

    pltpu.sync_copy(title_hbm.at[pl.ds(base * TL, CH * TL)], tidx_u)
    pltpu.sync_copy(body_hbm.at[pl.ds(base * BL, CH * BL)], bidx_u)
    _zero_acc(acc_t)
    _zero_acc(acc_b)

    lane = lax.iota(jnp.int32, 16)

    def tpass(k, _):
        for e0 in range(CH // 16):
            flat = (e0 * 16 + lane) * TL + k
            tidx_v[k, pl.ds(e0 * 16, 16)] = plsc.load_gather(tidx_u, [flat])
        pltpu.async_copy(w_hbm.at[tidx_v.at[k]], acc_t, sem_t, add=True)
        return 0

    def bpass(k, _):
        for e0 in range(CH // 16):
            flat = (e0 * 16 + lane) * BL + k
            bidx_v[k, pl.ds(e0 * 16, 16)] = plsc.load_gather(bidx_u, [flat])
        pltpu.async_copy(w_hbm.at[bidx_v.at[k]], acc_b, sem_b, add=True)
        return 0

    lax.fori_loop(0, TL, tpass, 0)
    lax.fori_loop(0, BL, bpass, 0)

    def tdrain(k, _):
        pltpu.make_async_copy(w_hbm.at[tidx_v.at[0]], acc_t, sem_t).wait()
        return 0

    def bdrain(k, _):
        pltpu.make_async_copy(w_hbm.at[bidx_v.at[0]], acc_b, sem_b).wait()
        return 0

    lax.fori_loop(0, TL, tdrain, 0)
    lax.fori_loop(0, BL, bdrain, 0)

    pltpu.sync_copy(acc_t, tsum_hbm.at[pl.ds(base, CH)])
    pltpu.sync_copy(acc_b, bsum_hbm.at[pl.ds(base, CH)])


_sc_pool = functools.partial(
    pl.kernel,
    out_type=(
        jax.ShapeDtypeStruct((N, D), jnp.float32),
        jax.ShapeDtypeStruct((N, D), jnp.float32),
    ),
    mesh=plsc.VectorSubcoreMesh(core_axis_name="c", subcore_axis_name="s"),
    scratch_types=[
        pltpu.VMEM((CH * TL,), jnp.int32),
        pltpu.VMEM((CH * BL,), jnp.int32),
        pltpu.VMEM((TL, CH), jnp.int32),
        pltpu.VMEM((BL, CH), jnp.int32),
        pltpu.VMEM((CH, D), jnp.float32),
        pltpu.VMEM((CH, D), jnp.float32),
        pltpu.SemaphoreType.DMA,
        pltpu.SemaphoreType.DMA,
    ],
    compiler_params=pltpu.CompilerParams(use_tc_tiling_on_sc=False,
                                         needs_layout_passes=False),
)(_sc_pool_body)


def _head_body(tidx_ref, bidx_ref, ts_ref, bs_ref, c_ref, o_ref):
    tcnt = jnp.sum((tidx_ref[...] > 0).astype(jnp.float32), axis=1, keepdims=True)
    bcnt = jnp.sum((bidx_ref[...] > 0).astype(jnp.float32), axis=1, keepdims=True)
    que = 0.3 * ts_ref[...] / tcnt + 0.7 * bs_ref[...] / bcnt
    sc = lax.dot_general(c_ref[...], que, (((1,), (1,)), ((), ())),
                         preferred_element_type=jnp.float32)
    m = jnp.max(sc, axis=0, keepdims=True)
    e = jnp.exp(sc - m)
    o_ref[...] = e / jnp.sum(e, axis=0, keepdims=True)


_R = 1024


def _head(tidx, bidx, tsum, bsum, c_table):
    return pl.pallas_call(
        _head_body,
        out_shape=jax.ShapeDtypeStruct((C, N), jnp.float32),
        grid=(N // _R,),
        in_specs=[
            pl.BlockSpec((_R, TL), lambda i: (i, 0)),
            pl.BlockSpec((_R, BL), lambda i: (i, 0)),
            pl.BlockSpec((_R, D), lambda i: (i, 0)),
            pl.BlockSpec((_R, D), lambda i: (i, 0)),
            pl.BlockSpec((C, D), lambda i: (0, 0)),
        ],
        out_specs=pl.BlockSpec((C, _R), lambda i: (0, i)),
    )(tidx, bidx, tsum, bsum, c_table)


def kernel(title_int, body_int, user_int, w_table, c_table):
    t = title_int.astype(jnp.int32)
    b = body_int.astype(jnp.int32)
    tsum, bsum = _sc_pool(w_table, t.reshape(-1), b.reshape(-1))
    return _head(t, b, tsum, bsum, c_table).T
